# baseline (speedup 1.0000x reference)
import jax
import jax.numpy as jnp
from jax.experimental import pallas as pl

_HIDDEN = 768
_EXPERTS = 64
_BLOCK_M = 2048


def _router_kernel(x_ref, w_ref, o_ref):
    o_ref[...] = jax.lax.dot_general(
        x_ref[...],
        w_ref[...],
        dimension_numbers=(((1,), (1,)), ((), ())),
        preferred_element_type=jnp.float32,
    )


@jax.jit
def kernel(hidden_states, W):
    x = hidden_states.reshape(-1, _HIDDEN)
    m = x.shape[0]
    grid = (m // _BLOCK_M,)
    return pl.pallas_call(
        _router_kernel,
        grid=grid,
        in_specs=[
            pl.BlockSpec((_BLOCK_M, _HIDDEN), lambda i: (i, 0)),
            pl.BlockSpec((_EXPERTS, _HIDDEN), lambda i: (0, 0)),
        ],
        out_specs=pl.BlockSpec((_BLOCK_M, _EXPERTS), lambda i: (i, 0)),
        out_shape=jax.ShapeDtypeStruct((m, _EXPERTS), jnp.float32),
    )(x, W)


# speedup vs baseline: 1.0216x; 1.0216x over previous
import jax
import jax.numpy as jnp
from jax.experimental import pallas as pl
from jax.experimental.pallas import tpu as pltpu

_HIDDEN = 768
_EXPERTS = 64
_BLOCK_M = 4096


def _router_kernel(x_ref, w_ref, o_ref):
    o_ref[...] = jax.lax.dot_general(
        x_ref[...],
        w_ref[...],
        dimension_numbers=(((1,), (1,)), ((), ())),
        preferred_element_type=jnp.float32,
    )


@jax.jit
def kernel(hidden_states, W):
    x = hidden_states.reshape(-1, _HIDDEN)
    m = x.shape[0]
    grid = (m // _BLOCK_M,)
    return pl.pallas_call(
        _router_kernel,
        grid=grid,
        in_specs=[
            pl.BlockSpec((_BLOCK_M, _HIDDEN), lambda i: (i, 0)),
            pl.BlockSpec((_EXPERTS, _HIDDEN), lambda i: (0, 0)),
        ],
        out_specs=pl.BlockSpec((_BLOCK_M, _EXPERTS), lambda i: (i, 0)),
        out_shape=jax.ShapeDtypeStruct((m, _EXPERTS), jnp.float32),
        compiler_params=pltpu.CompilerParams(
            dimension_semantics=("arbitrary",),
        ),
    )(x, W)
